# baseline (speedup 1.0000x reference)
import functools
import jax
import jax.numpy as jnp
from jax import lax
from jax.experimental import pallas as pl
from jax.experimental.pallas import tpu as pltpu
from jax.experimental.pallas import tpu_sc as plsc

MAX_LEN = 8192
D_MODEL = 768
BATCH = 4
DI = D_MODEL // 2

NC = 2
NS = 16
NW = NC * NS
L_PER_W = MAX_LEN // NW
K = 8
N_CHUNKS = L_PER_W // K
NSLOT = 4
NOSLOT = 2
GROUPS = N_CHUNKS // NSLOT
LANES = 16


def _make_kernel():
  mesh = plsc.VectorSubcoreMesh(core_axis_name="c", subcore_axis_name="s")

  @functools.partial(
      pl.kernel,
      out_type=jax.ShapeDtypeStruct((BATCH * MAX_LEN, D_MODEL), jnp.float32),
      mesh=mesh,
      compiler_params=pltpu.CompilerParams(needs_layout_passes=False),
      scratch_types=[
          pltpu.VMEM((BATCH, L_PER_W), jnp.int32),
          [pltpu.VMEM((K, D_MODEL), jnp.float32) for _ in range(NSLOT)],
          [[pltpu.VMEM((K, DI), jnp.int32) for _ in range(BATCH)]
           for _ in range(NSLOT)],
          [[pltpu.VMEM((K, D_MODEL), jnp.float32) for _ in range(BATCH)]
           for _ in range(NOSLOT)],
          [pltpu.SemaphoreType.DMA for _ in range(NSLOT)],
          [pltpu.SemaphoreType.DMA for _ in range(NSLOT)],
          pltpu.SemaphoreType.DMA,
      ],
  )
  def emb_kernel(x_hbm, table_hbm, pe_hbm, out_hbm, idx_v, pe_vs, gbufs,
                 obufs, psems, gsems, ssem):
    wid = lax.axis_index("s") * NC + lax.axis_index("c")
    l0 = wid * L_PER_W

    def idx_ref(c, b):
      return idx_v.at[b, pl.ds(c * K, K)]

    def fire(c, s):
      lc = l0 + c * K
      pltpu.async_copy(pe_hbm.at[pl.ds(lc, K)], pe_vs[s], psems[s])
      for b in range(BATCH):
        pltpu.async_copy(table_hbm.at[idx_ref(c, b)], gbufs[s][b], gsems[s])

    def wait_chunk(c, s):
      lc = l0 + c * K
      pltpu.make_async_copy(pe_hbm.at[pl.ds(lc, K)], pe_vs[s], psems[s]).wait()
      for b in range(BATCH):
        pltpu.make_async_copy(table_hbm.at[idx_ref(c, b)], gbufs[s][b],
                              gsems[s]).wait()

    def drain_stores(n):
      for _ in range(n):
        pltpu.make_async_copy(obufs[0][0], out_hbm.at[pl.ds(l0, K)],
                              ssem).wait()

    def process(c, s, ob):
      lc = l0 + c * K

      @pl.when(c >= NOSLOT)
      def _():
        drain_stores(BATCH)

      wait_chunk(c, s)
      pe_v = pe_vs[s]
      iota2 = lax.iota(jnp.int32, 16) * 2

      def row_body(r, _):
        rsplat = jnp.full((16,), r, jnp.int32)
        for j in range(DI // LANES):
          ce = iota2 + (j * 2 * LANES)
          co = ce + 1
          pe_e = plsc.load_gather(pe_v, [rsplat, ce])
          pe_o = plsc.load_gather(pe_v, [rsplat, co])
          for b in range(BATCH):
            vi = gbufs[s][b][r, pl.ds(j * LANES, LANES)]
            ve = lax.bitcast_convert_type(vi << 16, jnp.float32) + pe_e
            vo = lax.bitcast_convert_type(vi & jnp.int32(-65536),
                                          jnp.float32) + pe_o
            plsc.store_scatter(obufs[ob][b], [rsplat, ce], ve)
            plsc.store_scatter(obufs[ob][b], [rsplat, co], vo)
        return 0

      lax.fori_loop(0, K, row_body, 0)
      for b in range(BATCH):
        pltpu.async_copy(obufs[ob][b], out_hbm.at[pl.ds(b * MAX_LEN + lc, K)],
                         ssem)

    for b in range(BATCH):
      pltpu.sync_copy(x_hbm.at[pl.ds(b * MAX_LEN + l0, L_PER_W)], idx_v.at[b])

    fire(0, 0)
    fire(1, 1)

    def group_body(i, _):
      for k in range(NSLOT):
        c = NSLOT * i + k
        cn = c + 2

        @pl.when(cn < N_CHUNKS)
        def _():
          fire(cn, (k + 2) % NSLOT)

        process(c, k, k % NOSLOT)
      return 0

    lax.fori_loop(0, GROUPS, group_body, 0)
    drain_stores(NOSLOT * BATCH)

  return emb_kernel


_emb_kernel = _make_kernel()


@jax.jit
def kernel(x, table, pe):
  x_flat = x.reshape(BATCH * MAX_LEN).astype(jnp.int32)
  packed = lax.bitcast_convert_type(
      table.astype(jnp.bfloat16).reshape(MAX_LEN, DI, 2), jnp.int32)
  out = _emb_kernel(x_flat, packed, pe)
  return out.reshape(BATCH, MAX_LEN, D_MODEL)


# speedup vs baseline: 1.3232x; 1.3232x over previous
import functools
import jax
import jax.numpy as jnp
from jax import lax
from jax.experimental import pallas as pl
from jax.experimental.pallas import tpu as pltpu
from jax.experimental.pallas import tpu_sc as plsc

MAX_LEN = 8192
D_MODEL = 768
BATCH = 4
DI = D_MODEL // 2

NC = 2
NS = 16
NW = NC * NS
L_PER_W = MAX_LEN // NW
K = 8
N_CHUNKS = L_PER_W // K
NSLOT = 4
NOSLOT = 2
GROUPS = N_CHUNKS // NSLOT
LANES = 16


def _make_kernel():
  mesh = plsc.VectorSubcoreMesh(core_axis_name="c", subcore_axis_name="s")

  @functools.partial(
      pl.kernel,
      out_type=jax.ShapeDtypeStruct((BATCH * MAX_LEN, D_MODEL), jnp.float32),
      mesh=mesh,
      scratch_types=[
          pltpu.VMEM((BATCH, L_PER_W), jnp.int32),
          [pltpu.VMEM((K, D_MODEL), jnp.float32) for _ in range(2)],
          [[pltpu.VMEM((K, DI), jnp.int32) for _ in range(BATCH)]
           for _ in range(NSLOT)],
          [[pltpu.VMEM((K, D_MODEL), jnp.float32) for _ in range(BATCH)]
           for _ in range(NOSLOT)],
          [pltpu.SemaphoreType.DMA for _ in range(2)],
          [pltpu.SemaphoreType.DMA for _ in range(NSLOT)],
          pltpu.SemaphoreType.DMA,
      ],
  )
  def emb_kernel(x_hbm, table_hbm, pe_hbm, out_hbm, idx_v, pe_vs, gbufs,
                 obufs, psems, gsems, ssem):
    wid = lax.axis_index("s") * NC + lax.axis_index("c")
    l0 = wid * L_PER_W

    def idx_ref(c, b):
      return idx_v.at[b, pl.ds(c * K, K)]

    def fire_pe(c, ps):
      lc = l0 + c * K
      pltpu.async_copy(pe_hbm.at[pl.ds(lc, K)], pe_vs[ps], psems[ps])

    def fire(c, s):
      lc = l0 + c * K
      for b in range(BATCH):
        pltpu.async_copy(table_hbm.at[idx_ref(c, b)], gbufs[s][b], gsems[s])

    def wait_chunk(c, s, ps):
      lc = l0 + c * K
      pltpu.make_async_copy(pe_hbm.at[pl.ds(lc, K)], pe_vs[ps],
                            psems[ps]).wait()
      for b in range(BATCH):
        pltpu.make_async_copy(table_hbm.at[idx_ref(c, b)], gbufs[s][b],
                              gsems[s]).wait()

    def drain_stores(n):
      for _ in range(n):
        pltpu.make_async_copy(obufs[0][0], out_hbm.at[pl.ds(l0, K)],
                              ssem).wait()

    def process(c, s, ob, ps):
      lc = l0 + c * K

      @pl.when(c >= NOSLOT)
      def _():
        drain_stores(BATCH)

      wait_chunk(c, s, ps)
      pe_v = pe_vs[ps]

      def row_body(r, _):
        def j_body(jo, _2):
          for ji in range(4):
            jbase = jo * 4 * LANES + ji * LANES
            lo = 2 * jbase
            hi = lo + LANES
            pe_a = pe_v[r, pl.ds(lo, LANES)]
            pe_b = pe_v[r, pl.ds(hi, LANES)]
            for b in range(BATCH):
              vi = gbufs[s][b][r, pl.ds(jbase, LANES)]
              va = lax.bitcast_convert_type(vi << 16, jnp.float32) + pe_a
              vb = lax.bitcast_convert_type(vi & jnp.int32(-65536),
                                            jnp.float32) + pe_b
              obufs[ob][b][r, pl.ds(lo, LANES)] = va
              obufs[ob][b][r, pl.ds(hi, LANES)] = vb
          return 0

        lax.fori_loop(0, (DI // LANES) // 4, j_body, 0)
        return 0

      lax.fori_loop(0, K, row_body, 0)
      for b in range(BATCH):
        pltpu.async_copy(obufs[ob][b], out_hbm.at[pl.ds(b * MAX_LEN + lc, K)],
                         ssem)

    for b in range(BATCH):
      pltpu.sync_copy(x_hbm.at[pl.ds(b * MAX_LEN + l0, L_PER_W)], idx_v.at[b])

    fire(0, 0)
    fire(1, 1)
    fire_pe(0, 0)

    def group_body(i, _):
      for k in range(NSLOT):
        c = NSLOT * i + k
        cn = c + 2

        @pl.when(cn < N_CHUNKS)
        def _():
          fire(cn, (k + 2) % NSLOT)

        @pl.when(c + 1 < N_CHUNKS)
        def _():
          fire_pe(c + 1, (k + 1) % 2)

        process(c, k, k % NOSLOT, k % 2)
      return 0

    lax.fori_loop(0, GROUPS, group_body, 0)
    drain_stores(NOSLOT * BATCH)

  return emb_kernel


_emb_kernel = _make_kernel()


@jax.jit
def kernel(x, table, pe):
  x_flat = x.reshape(BATCH * MAX_LEN).astype(jnp.int32)
  tb = table.astype(jnp.bfloat16).reshape(MAX_LEN, DI // LANES, 2, LANES)
  paired = jnp.stack([tb[:, :, 0, :], tb[:, :, 1, :]], axis=-1)
  packed = lax.bitcast_convert_type(paired, jnp.int32).reshape(MAX_LEN, DI)
  out = _emb_kernel(x_flat, packed, pe)
  return out.reshape(BATCH, MAX_LEN, D_MODEL)


# speedup vs baseline: 2.8741x; 2.1720x over previous
import functools
import jax
import jax.numpy as jnp
from jax import lax
from jax.experimental import pallas as pl
from jax.experimental.pallas import tpu as pltpu
from jax.experimental.pallas import tpu_sc as plsc

MAX_LEN = 8192
D_MODEL = 768
BATCH = 4

NC = 2
NS = 16
NW = NC * NS
L_PER_W = MAX_LEN // NW
K = 8
RPC = BATCH * K
N_CHUNKS = L_PER_W // K
NSLOT = 4
GROUPS = N_CHUNKS // NSLOT
LANES = 16


def _make_kernel():
  mesh = plsc.VectorSubcoreMesh(core_axis_name="c", subcore_axis_name="s")

  @functools.partial(
      pl.kernel,
      out_type=jax.ShapeDtypeStruct((BATCH * MAX_LEN, D_MODEL), jnp.float32),
      mesh=mesh,
      scratch_types=[
          pltpu.VMEM((N_CHUNKS, RPC), jnp.int32),
          pltpu.VMEM((N_CHUNKS, RPC), jnp.int32),
          [pltpu.VMEM((RPC, D_MODEL), jnp.float32) for _ in range(NSLOT)],
          [pltpu.VMEM((K, D_MODEL), jnp.float32) for _ in range(NSLOT)],
          [pltpu.SemaphoreType.DMA for _ in range(NSLOT)],
          [pltpu.SemaphoreType.DMA for _ in range(NSLOT)],
          pltpu.SemaphoreType.DMA,
      ],
  )
  def emb_kernel(gidx_hbm, oidx_hbm, table_hbm, pe_hbm, out_hbm, gidx_v,
                 oidx_v, row_vs, pe_vs, psems, gsems, ssem):
    wid = lax.axis_index("s") * NC + lax.axis_index("c")
    l0 = wid * L_PER_W

    def fire(c, s):
      lc = l0 + c * K
      pltpu.async_copy(pe_hbm.at[pl.ds(lc, K)], pe_vs[s], psems[s])
      pltpu.async_copy(table_hbm.at[gidx_v.at[c]], row_vs[s], gsems[s])

    def wait_chunk(c, s):
      lc = l0 + c * K
      pltpu.make_async_copy(pe_hbm.at[pl.ds(lc, K)], pe_vs[s], psems[s]).wait()
      pltpu.make_async_copy(table_hbm.at[gidx_v.at[c]], row_vs[s],
                            gsems[s]).wait()

    def drain_scatters(n):
      for _ in range(n):
        pltpu.make_async_copy(row_vs[0], out_hbm.at[oidx_v.at[0]],
                              ssem).wait()

    def process(c, s):
      wait_chunk(c, s)
      pe_v = pe_vs[s]

      def row_body(r, _):
        for j in range(D_MODEL // LANES):
          sl = pl.ds(j * LANES, LANES)
          pv = pe_v[r, sl]
          for b in range(BATCH):
            row_vs[s][b * K + r, sl] = row_vs[s][b * K + r, sl] + pv
        return 0

      lax.fori_loop(0, K, row_body, 0)
      pltpu.async_copy(row_vs[s], out_hbm.at[oidx_v.at[c]], ssem)

    pltpu.sync_copy(gidx_hbm.at[wid], gidx_v)
    pltpu.sync_copy(oidx_hbm.at[wid], oidx_v)

    fire(0, 0)
    fire(1, 1)

    def group_body(i, _):
      for k in range(NSLOT):
        c = NSLOT * i + k
        cn = c + 2

        @pl.when(jnp.logical_and(cn >= NSLOT, cn < N_CHUNKS))
        def _():
          drain_scatters(1)

        @pl.when(cn < N_CHUNKS)
        def _():
          fire(cn, (k + 2) % NSLOT)

        process(c, k)
      return 0

    lax.fori_loop(0, GROUPS, group_body, 0)
    drain_scatters(NSLOT)

  return emb_kernel


_emb_kernel = _make_kernel()


@jax.jit
def kernel(x, table, pe):
  gidx = (x.astype(jnp.int32)
          .reshape(BATCH, NW, N_CHUNKS, K)
          .transpose(1, 2, 0, 3)
          .reshape(NW, N_CHUNKS, RPC))
  b_ids = jnp.arange(BATCH, dtype=jnp.int32) * MAX_LEN
  l_ids = (jnp.arange(NW, dtype=jnp.int32)[:, None, None] * L_PER_W
           + jnp.arange(N_CHUNKS, dtype=jnp.int32)[None, :, None] * K
           + jnp.arange(K, dtype=jnp.int32)[None, None, :])
  oidx = (l_ids[:, :, None, :] + b_ids[None, None, :, None]).reshape(
      NW, N_CHUNKS, RPC)
  out = _emb_kernel(gidx, oidx, table, pe)
  return out.reshape(BATCH, MAX_LEN, D_MODEL)
